# Initial kernel scaffold; baseline (speedup 1.0000x reference)
import jax
import jax.numpy as jnp
from jax.experimental import pallas as pl


def kernel(x, logit):
    raise NotImplementedError("write your pallas kernel here")


# speedup vs baseline: 1.5528x; 1.5528x over previous
import jax
import jax.numpy as jnp
from jax.experimental import pallas as pl
from jax.experimental.pallas import tpu as pltpu

_P = 32768
_K = 1024
_ROWS = 16
_SUB = 256


def _count_ge(u, cand):
    return jnp.sum((u >= cand).astype(jnp.float32))


def _gate_kernel(l2_ref, lrow_ref, x_ref, o_ref, mask_ref):
    @pl.when(pl.program_id(0) == 0)
    def _compute_mask():
        prob2 = jax.nn.sigmoid(l2_ref[...])
        u2 = jax.lax.bitcast_convert_type(prob2, jnp.int32)
        base = jnp.int32(0)
        for b in range(30, -1, -1):
            cand = base + jnp.int32(1 << b)
            base = jnp.where(_count_ge(u2, cand) >= _K, cand, base)
        t = base
        cnt_gt = jnp.sum((u2 > t).astype(jnp.float32))
        need = jnp.float32(_K) - cnt_gt
        eq2 = (u2 == t)
        idx2 = (jax.lax.broadcasted_iota(jnp.int32, (_SUB, 128), 0) * 128
                + jax.lax.broadcasted_iota(jnp.int32, (_SUB, 128), 1))
        m = jnp.int32(0)
        for b in range(14, -1, -1):
            cand = m + jnp.int32(1 << b)
            c = jnp.sum((eq2 & (idx2 < cand)).astype(jnp.float32))
            m = jnp.where(c < need, cand, m)
        probr = jax.nn.sigmoid(lrow_ref[...])
        ur = jax.lax.bitcast_convert_type(probr, jnp.int32)
        idxr = jax.lax.broadcasted_iota(jnp.int32, (1, _P), 1)
        mask_ref[...] = ((ur > t) | ((ur == t) & (idxr <= m))).astype(
            jnp.float32)

    o_ref[...] = x_ref[...] * mask_ref[...]


def kernel(x, logit):
    l2 = logit.reshape(_SUB, 128)
    lrow = logit.reshape(1, _P)
    return pl.pallas_call(
        _gate_kernel,
        grid=(x.shape[0] // _ROWS,),
        in_specs=[
            pl.BlockSpec((_SUB, 128), lambda i: (0, 0)),
            pl.BlockSpec((1, _P), lambda i: (0, 0)),
            pl.BlockSpec((_ROWS, _P), lambda i: (i, 0)),
        ],
        out_specs=pl.BlockSpec((_ROWS, _P), lambda i: (i, 0)),
        out_shape=jax.ShapeDtypeStruct(x.shape, x.dtype),
        scratch_shapes=[pltpu.VMEM((1, _P), jnp.float32)],
    )(l2, lrow, x)


# speedup vs baseline: 1.8214x; 1.1729x over previous
import jax
import jax.numpy as jnp
from jax.experimental import pallas as pl
from jax.experimental.pallas import tpu as pltpu

_P = 32768
_K = 1024
_ROWS = 16
_SUB = 256


def _gate_kernel(l2_ref, lrow_ref, x_ref, o_ref, mask_ref):
    @pl.when(pl.program_id(0) == 0)
    def _compute_mask():
        prob2 = jax.nn.sigmoid(l2_ref[...])
        u2 = jax.lax.bitcast_convert_type(prob2, jnp.int32)
        base = jnp.int32(0)
        for s in [1 << b for b in range(28, 0, -3)] + [1]:
            step = jnp.int32(0)
            for j in range(1, 8):
                c = jnp.sum((u2 >= base + jnp.int32(j * s)).astype(
                    jnp.float32))
                step = step + jnp.where(c >= _K, jnp.int32(s), jnp.int32(0))
            base = base + step
        t = base
        cnt_gt = jnp.sum((u2 > t).astype(jnp.float32))
        need = jnp.float32(_K) - cnt_gt
        eq2 = (u2 == t)
        idx2 = (jax.lax.broadcasted_iota(jnp.int32, (_SUB, 128), 0) * 128
                + jax.lax.broadcasted_iota(jnp.int32, (_SUB, 128), 1))
        ef2 = eq2.astype(jnp.float32)
        m = jnp.int32(0)
        for s in (4096, 512, 64, 8, 1):
            step = jnp.int32(0)
            for j in range(1, 8):
                c = jnp.sum(jnp.where(idx2 < m + jnp.int32(j * s), ef2, 0.0))
                step = step + jnp.where(c < need, jnp.int32(s), jnp.int32(0))
            m = m + step
        probr = jax.nn.sigmoid(lrow_ref[...])
        ur = jax.lax.bitcast_convert_type(probr, jnp.int32)
        idxr = jax.lax.broadcasted_iota(jnp.int32, (1, _P), 1)
        mask_ref[...] = ((ur > t) | ((ur == t) & (idxr <= m))).astype(
            jnp.float32)

    o_ref[...] = x_ref[...] * mask_ref[...]


def kernel(x, logit):
    l2 = logit.reshape(_SUB, 128)
    lrow = logit.reshape(1, _P)
    return pl.pallas_call(
        _gate_kernel,
        grid=(x.shape[0] // _ROWS,),
        in_specs=[
            pl.BlockSpec((_SUB, 128), lambda i: (0, 0)),
            pl.BlockSpec((1, _P), lambda i: (0, 0)),
            pl.BlockSpec((_ROWS, _P), lambda i: (i, 0)),
        ],
        out_specs=pl.BlockSpec((_ROWS, _P), lambda i: (i, 0)),
        out_shape=jax.ShapeDtypeStruct(x.shape, x.dtype),
        scratch_shapes=[pltpu.VMEM((1, _P), jnp.float32)],
    )(l2, lrow, x)


# speedup vs baseline: 1.9450x; 1.0679x over previous
import jax
import jax.numpy as jnp
from jax.experimental import pallas as pl
from jax.experimental.pallas import tpu as pltpu

_P = 32768
_K = 1024
_ROWS = 32
_SUB = 256


def _gate_kernel(l2_ref, lrow_ref, x_ref, o_ref, mask_ref):
    @pl.when(pl.program_id(0) == 0)
    def _compute_mask():
        prob2 = jax.nn.sigmoid(l2_ref[...])
        u2 = jax.lax.bitcast_convert_type(prob2, jnp.int32)
        base = jnp.int32(0)
        for s in [1 << b for b in range(28, 0, -3)] + [1]:
            step = jnp.int32(0)
            for j in range(1, 8):
                c = jnp.sum((u2 >= base + jnp.int32(j * s)).astype(
                    jnp.float32))
                step = step + jnp.where(c >= _K, jnp.int32(s), jnp.int32(0))
            base = base + step
        t = base
        cnt_gt = jnp.sum((u2 > t).astype(jnp.float32))
        need = jnp.float32(_K) - cnt_gt
        eq2 = (u2 == t)
        idx2 = (jax.lax.broadcasted_iota(jnp.int32, (_SUB, 128), 0) * 128
                + jax.lax.broadcasted_iota(jnp.int32, (_SUB, 128), 1))
        ef2 = eq2.astype(jnp.float32)
        m = jnp.int32(0)
        for s in (4096, 512, 64, 8, 1):
            step = jnp.int32(0)
            for j in range(1, 8):
                c = jnp.sum(jnp.where(idx2 < m + jnp.int32(j * s), ef2, 0.0))
                step = step + jnp.where(c < need, jnp.int32(s), jnp.int32(0))
            m = m + step
        probr = jax.nn.sigmoid(lrow_ref[...])
        ur = jax.lax.bitcast_convert_type(probr, jnp.int32)
        idxr = jax.lax.broadcasted_iota(jnp.int32, (1, _P), 1)
        mask_ref[...] = ((ur > t) | ((ur == t) & (idxr <= m))).astype(
            jnp.float32)

    o_ref[...] = x_ref[...] * mask_ref[...]


def kernel(x, logit):
    l2 = logit.reshape(_SUB, 128)
    lrow = logit.reshape(1, _P)
    return pl.pallas_call(
        _gate_kernel,
        grid=(x.shape[0] // _ROWS,),
        in_specs=[
            pl.BlockSpec((_SUB, 128), lambda i: (0, 0)),
            pl.BlockSpec((1, _P), lambda i: (0, 0)),
            pl.BlockSpec((_ROWS, _P), lambda i: (i, 0)),
        ],
        out_specs=pl.BlockSpec((_ROWS, _P), lambda i: (i, 0)),
        out_shape=jax.ShapeDtypeStruct(x.shape, x.dtype),
        scratch_shapes=[pltpu.VMEM((1, _P), jnp.float32)],
    )(l2, lrow, x)


# speedup vs baseline: 2.2192x; 1.1410x over previous
import jax
import jax.numpy as jnp
from jax.experimental import pallas as pl
from jax.experimental.pallas import tpu as pltpu

_P = 32768
_K = 1024
_ROWS = 64
_SUB = 256


def _gate_kernel(l2_ref, lrow_ref, x_ref, o_ref, mask_ref):
    @pl.when(pl.program_id(0) == 0)
    def _compute_mask():
        prob2 = jax.nn.sigmoid(l2_ref[...])
        u2 = jax.lax.bitcast_convert_type(prob2, jnp.int32)
        base = jnp.int32(0)
        for s in [1 << b for b in range(28, 0, -3)] + [1]:
            step = jnp.int32(0)
            for j in range(1, 8):
                c = jnp.sum((u2 >= base + jnp.int32(j * s)).astype(
                    jnp.float32))
                step = step + jnp.where(c >= _K, jnp.int32(s), jnp.int32(0))
            base = base + step
        t = base
        cnt_gt = jnp.sum((u2 > t).astype(jnp.float32))
        need = jnp.float32(_K) - cnt_gt
        eq2 = (u2 == t)
        idx2 = (jax.lax.broadcasted_iota(jnp.int32, (_SUB, 128), 0) * 128
                + jax.lax.broadcasted_iota(jnp.int32, (_SUB, 128), 1))
        ef2 = eq2.astype(jnp.float32)
        m = jnp.int32(0)
        for s in (4096, 512, 64, 8, 1):
            step = jnp.int32(0)
            for j in range(1, 8):
                c = jnp.sum(jnp.where(idx2 < m + jnp.int32(j * s), ef2, 0.0))
                step = step + jnp.where(c < need, jnp.int32(s), jnp.int32(0))
            m = m + step
        probr = jax.nn.sigmoid(lrow_ref[...])
        ur = jax.lax.bitcast_convert_type(probr, jnp.int32)
        idxr = jax.lax.broadcasted_iota(jnp.int32, (1, _P), 1)
        mask_ref[...] = ((ur > t) | ((ur == t) & (idxr <= m))).astype(
            jnp.float32)

    o_ref[...] = x_ref[...] * mask_ref[...]


def kernel(x, logit):
    l2 = logit.reshape(_SUB, 128)
    lrow = logit.reshape(1, _P)
    return pl.pallas_call(
        _gate_kernel,
        grid=(x.shape[0] // _ROWS,),
        in_specs=[
            pl.BlockSpec((_SUB, 128), lambda i: (0, 0)),
            pl.BlockSpec((1, _P), lambda i: (0, 0)),
            pl.BlockSpec((_ROWS, _P), lambda i: (i, 0)),
        ],
        out_specs=pl.BlockSpec((_ROWS, _P), lambda i: (i, 0)),
        out_shape=jax.ShapeDtypeStruct(x.shape, x.dtype),
        scratch_shapes=[pltpu.VMEM((1, _P), jnp.float32)],
    )(l2, lrow, x)
